# baseline (speedup 1.0000x reference)
import jax
import jax.numpy as jnp
from jax.experimental import pallas as pl
from jax.experimental.pallas import tpu as pltpu

_BF16 = jnp.bfloat16
_VLIM = int(58 * 2**20)
_BB = 16
_P0R = 312


def _cparams(sem):
    return pltpu.CompilerParams(dimension_semantics=sem, vmem_limit_bytes=_VLIM)


def _conv_kernel(a_ref, m_ref, w0_ref, b0_ref, w1_ref, b1_ref, w2_ref, b2_ref,
                 w3_ref, b3_ref, o_ref, lhsT, p0d, p0, l0h, y1p, p1, l1h, y2p,
                 l2h):
    b0 = b0_ref[...]
    msk = m_ref[...]
    lhsT[27:32, :] = jnp.zeros((5, 1280), _BF16)
    for i in range(_BB):
        for t in range(9):
            dy, dx = t // 3, t % 3
            sh = dy * 36 + dx
            lhsT[t * 3:(t + 1) * 3, :] = a_ref[i, :, sh:sh + 1280]
        h = jnp.dot(lhsT[...].T, w0_ref[...],
                    preferred_element_type=jnp.float32)
        y = jnp.maximum(h[:, :128] + h[:, 128:] + b0, 0.0)
        p0d[i * 1280:(i + 1) * 1280, :] = y * msk

    p0[...] = jnp.zeros_like(p0)
    for i in range(_BB):
        for pln in range(4):
            pp, q = pln // 2, pln % 2
            for ii in range(17):
                base = i * 1280 + (2 * ii + pp) * 36 + q
                dst = (i * 4 + pln) * _P0R + ii * 17
                p0[dst:dst + 17, :] = (
                    p0d[pl.Slice(base, 17, 2), :].astype(_BF16))

    for i in range(_BB):
        for t in range(9):
            dy, dx = t // 3, t % 3
            pln = (dy % 2) * 2 + (dx % 2)
            st = (dy // 2) * 17 + dx // 2
            src = (i * 4 + pln) * _P0R + st
            l0h[i * 272:(i + 1) * 272, t * 128:(t + 1) * 128] = (
                p0[src:src + 272, :])
    b1 = b1_ref[...]
    y1p[...] = jnp.zeros_like(y1p)
    for s in range(2):
        h = jnp.dot(l0h[s * (_BB // 2) * 272:(s + 1) * (_BB // 2) * 272],
                    w1_ref[...],
                    preferred_element_type=jnp.float32)
        y = jnp.maximum(h[:, :128] + h[:, 128:] + b1, 0.0)
        for im in range(_BB // 2):
            i = s * (_BB // 2) + im
            for yo in range(16):
                y1p[i * 360 + (yo + 1) * 18 + 1:i * 360 + (yo + 1) * 18 + 17,
                    :] = y[im * 272 + yo * 17:im * 272 + yo * 17 + 16, :]

    for i in range(_BB):
        for pln in range(4):
            pp, q = pln // 2, pln % 2
            for ii in range(10):
                base = i * 360 + (2 * ii + pp) * 18 + q
                p1[i, pln, ii * 9:ii * 9 + 9, :] = (
                    y1p[pl.Slice(base, 9, 2), :].astype(_BF16))

    for i in range(_BB):
        for t in range(9):
            dy, dx = t // 3, t % 3
            pln = (dy % 2) * 2 + (dx % 2)
            st = (dy // 2) * 9 + dx // 2
            l1h[i * 72:(i + 1) * 72, t * 128:(t + 1) * 128] = (
                p1[i, pln, st:st + 72, :])
    b2 = b2_ref[...]
    y2p[...] = jnp.zeros_like(y2p)
    for s in range(2):
        h = jnp.dot(l1h[s * (_BB // 2) * 72:(s + 1) * (_BB // 2) * 72],
                    w2_ref[...],
                    preferred_element_type=jnp.float32)
        y = jnp.maximum(h + b2, 0.0).astype(_BF16)
        for im in range(_BB // 2):
            i = s * (_BB // 2) + im
            for yo in range(8):
                y2p[i, (yo + 1) * 10 + 1:(yo + 1) * 10 + 9, :] = (
                    y[im * 72 + yo * 9:im * 72 + yo * 9 + 8, :])

    for i in range(_BB):
        for t in range(4):
            p, q = t // 2, t % 2
            l2h[i * 96:i * 96 + 91, t * 256:(t + 1) * 256] = (
                y2p[i, p * 10 + q:p * 10 + q + 91, :])
    b3 = b3_ref[...]
    for g in range(4):
        ga, gb = g // 2, g % 2
        h = jnp.dot(l2h[...], w3_ref[g], preferred_element_type=jnp.float32)
        y = jnp.maximum(h + b3, 0.0).astype(_BF16)
        for i in range(_BB):
            base = i * 96 + ga * 10 + gb
            blk = y[base:base + 80].reshape(8, 10, 256)[:, :8, :]
            o_ref[i, :, ga, :, gb * 256:(gb + 1) * 256] = blk


def _conv_call(a0, mask, rhs0, b0, rhs1, b1, w1, b1c, w2, b2c):
    n = a0.shape[0]
    return pl.pallas_call(
        _conv_kernel,
        out_shape=jax.ShapeDtypeStruct((n, 8, 2, 8, 512), _BF16),
        grid=(n // _BB,),
        in_specs=[
            pl.BlockSpec((_BB, 3, 1408), lambda b: (b, 0, 0)),
            pl.BlockSpec((1280, 128), lambda b: (0, 0)),
            pl.BlockSpec((32, 256), lambda b: (0, 0)),
            pl.BlockSpec((1, 128), lambda b: (0, 0)),
            pl.BlockSpec((1152, 256), lambda b: (0, 0)),
            pl.BlockSpec((1, 128), lambda b: (0, 0)),
            pl.BlockSpec((1152, 256), lambda b: (0, 0)),
            pl.BlockSpec((1, 256), lambda b: (0, 0)),
            pl.BlockSpec((4, 1024, 256), lambda b: (0, 0, 0)),
            pl.BlockSpec((1, 256), lambda b: (0, 0)),
        ],
        out_specs=pl.BlockSpec((_BB, 8, 2, 8, 512), lambda b: (b, 0, 0, 0, 0)),
        scratch_shapes=[
            pltpu.VMEM((32, 1280), _BF16),
            pltpu.VMEM((_BB * 1280, 128), jnp.float32),
            pltpu.VMEM((_BB * 4 * _P0R, 128), _BF16),
            pltpu.VMEM((_BB * 272, 1152), _BF16),
            pltpu.VMEM((_BB * 360, 128), jnp.float32),
            pltpu.VMEM((_BB, 4, 96, 128), _BF16),
            pltpu.VMEM((_BB * 72, 1152), _BF16),
            pltpu.VMEM((_BB, 110, 256), _BF16),
            pltpu.VMEM((_BB * 96, 1024), _BF16),
        ],
        compiler_params=_cparams(("parallel",)),
    )(a0, mask, rhs0, b0, rhs1, b1, w1, b1c, w2, b2c)


def _fc_kernel(a_ref, w1_ref, b1_ref, w2_ref, o_ref, acc_ref):
    i = pl.program_id(1)

    @pl.when(i == 0)
    def _():
        acc_ref[...] = jnp.zeros_like(acc_ref)

    upd = jnp.zeros_like(acc_ref)
    for s in range(8):
        upd += jnp.dot(a_ref[:, 0, 0, s, :], w1_ref[s * 512:(s + 1) * 512],
                       preferred_element_type=jnp.float32)
    acc_ref[...] += upd

    @pl.when(i == pl.num_programs(1) - 1)
    def _():
        h = jnp.maximum(acc_ref[...] + b1_ref[...], 0.0).astype(_BF16)
        o_ref[0] = jnp.dot(h, w2_ref[...], preferred_element_type=jnp.float32)


def _fc_call(a, w1, b1, w2):
    mp = a.shape[0]
    tk = 4096
    return pl.pallas_call(
        _fc_kernel,
        out_shape=jax.ShapeDtypeStruct((2, mp, 128), jnp.float32),
        grid=(2, 16),
        in_specs=[
            pl.BlockSpec((mp, 1, 1, 8, 512),
                         lambda j, i: (0, i // 2, i % 2, 0, 0)),
            pl.BlockSpec((tk, 256), lambda j, i: (i, j)),
            pl.BlockSpec((1, 256), lambda j, i: (0, j)),
            pl.BlockSpec((256, 128), lambda j, i: (j, 0)),
        ],
        out_specs=pl.BlockSpec((1, mp, 128), lambda j, i: (j, 0, 0)),
        scratch_shapes=[pltpu.VMEM((mp, 256), jnp.float32)],
        compiler_params=_cparams(("parallel", "arbitrary")),
    )(a, w1, b1, w2)


def kernel(x, init_w, init_b, layer0_w, layer0_b, layer1_w, layer1_b,
           layer2_w, layer2_b, fc1_w, fc1_b, fc2_w, fc2_b):
    n = x.shape[0]

    xb = jnp.pad(x.astype(_BF16), ((0, 0), (0, 0), (2, 2), (2, 2)))
    a0 = jnp.pad(xb.reshape(n, 3, 1296), ((0, 0), (0, 0), (0, 112)))

    r = jnp.arange(1280)
    u, v = r // 36, r % 36
    ok = (u >= 1) & (u <= 32) & (v >= 1) & (v <= 32)
    mask = jnp.broadcast_to(ok[:, None], (1280, 128)).astype(jnp.float32)

    w27 = init_w.reshape(9, 128, 128)[:, :3, :].reshape(27, 128)
    k27 = jnp.arange(27)[:, None]
    rhs0 = jnp.concatenate(
        [jnp.where(k27 < 15, w27, 0), jnp.where(k27 >= 15, w27, 0)], axis=1)
    rhs0 = jnp.pad(rhs0, ((0, 5), (0, 0)))

    w0 = layer0_w[0]
    kk = jnp.arange(1152)[:, None]
    rhs1 = jnp.concatenate(
        [jnp.where(kk < 640, w0, 0), jnp.where(kk >= 640, w0, 0)], axis=1)

    y3 = _conv_call(a0, mask, rhs0, init_b, rhs1, layer0_b,
                    layer1_w[0], layer1_b, layer2_w, layer2_b)

    parts = _fc_call(y3, fc1_w, fc1_b, fc2_w)
    out = parts[0] + parts[1] + fc2_b
    return out[:, :10]


# speedup vs baseline: 1.0374x; 1.0374x over previous
import jax
import jax.numpy as jnp
from jax.experimental import pallas as pl
from jax.experimental.pallas import tpu as pltpu

_BF16 = jnp.bfloat16
_VLIM = int(56 * 2**20)
_BB = 8
_P0R = 312


def _cparams(sem):
    return pltpu.CompilerParams(dimension_semantics=sem, vmem_limit_bytes=_VLIM)


def _conv_kernel(a_ref, m_ref, w0_ref, b0_ref, w1_ref, b1_ref, w2_ref, b2_ref,
                 w3_ref, b3_ref, o_ref, lhsT, p0d, p0, l0h, y1p, p1, l1h, y2p,
                 l2h):
    b0 = b0_ref[...]
    msk = m_ref[...]
    lhsT[27:32, :] = jnp.zeros((5, 1280), _BF16)
    for i in range(_BB):
        for t in range(9):
            dy, dx = t // 3, t % 3
            sh = dy * 36 + dx
            lhsT[t * 3:(t + 1) * 3, :] = a_ref[i, :, sh:sh + 1280]
        h = jnp.dot(lhsT[...].T, w0_ref[...],
                    preferred_element_type=jnp.float32)
        y = jnp.maximum(h[:, :128] + h[:, 128:] + b0, 0.0)
        p0d[i * 1280:(i + 1) * 1280, :] = y * msk

    p0[...] = jnp.zeros_like(p0)
    for i in range(_BB):
        for pln in range(4):
            pp, q = pln // 2, pln % 2
            for ii in range(17):
                base = i * 1280 + (2 * ii + pp) * 36 + q
                dst = (i * 4 + pln) * _P0R + ii * 17
                p0[dst:dst + 17, :] = (
                    p0d[pl.Slice(base, 17, 2), :].astype(_BF16))

    for i in range(_BB):
        for t in range(9):
            dy, dx = t // 3, t % 3
            pln = (dy % 2) * 2 + (dx % 2)
            st = (dy // 2) * 17 + dx // 2
            src = (i * 4 + pln) * _P0R + st
            l0h[i * 272:(i + 1) * 272, t * 128:(t + 1) * 128] = (
                p0[src:src + 272, :])
    b1 = b1_ref[...]
    y1p[...] = jnp.zeros_like(y1p)
    for s in range(2):
        h = jnp.dot(l0h[s * (_BB // 2) * 272:(s + 1) * (_BB // 2) * 272],
                    w1_ref[...],
                    preferred_element_type=jnp.float32)
        y = jnp.maximum(h[:, :128] + h[:, 128:] + b1, 0.0)
        for im in range(_BB // 2):
            i = s * (_BB // 2) + im
            for yo in range(16):
                y1p[i * 360 + (yo + 1) * 18 + 1:i * 360 + (yo + 1) * 18 + 17,
                    :] = y[im * 272 + yo * 17:im * 272 + yo * 17 + 16, :]

    for i in range(_BB):
        for pln in range(4):
            pp, q = pln // 2, pln % 2
            for ii in range(10):
                base = i * 360 + (2 * ii + pp) * 18 + q
                p1[i, pln, ii * 9:ii * 9 + 9, :] = (
                    y1p[pl.Slice(base, 9, 2), :].astype(_BF16))

    for i in range(_BB):
        for t in range(9):
            dy, dx = t // 3, t % 3
            pln = (dy % 2) * 2 + (dx % 2)
            st = (dy // 2) * 9 + dx // 2
            l1h[i * 72:(i + 1) * 72, t * 128:(t + 1) * 128] = (
                p1[i, pln, st:st + 72, :])
    b2 = b2_ref[...]
    y2p[...] = jnp.zeros_like(y2p)
    for s in range(2):
        h = jnp.dot(l1h[s * (_BB // 2) * 72:(s + 1) * (_BB // 2) * 72],
                    w2_ref[...],
                    preferred_element_type=jnp.float32)
        y = jnp.maximum(h + b2, 0.0).astype(_BF16)
        for im in range(_BB // 2):
            i = s * (_BB // 2) + im
            for yo in range(8):
                y2p[i, (yo + 1) * 10 + 1:(yo + 1) * 10 + 9, :] = (
                    y[im * 72 + yo * 9:im * 72 + yo * 9 + 8, :])

    for i in range(_BB):
        for t in range(4):
            p, q = t // 2, t % 2
            l2h[i * 96:i * 96 + 91, t * 256:(t + 1) * 256] = (
                y2p[i, p * 10 + q:p * 10 + q + 91, :])
    b3 = b3_ref[...]
    for g in range(4):
        ga, gb = g // 2, g % 2
        h = jnp.dot(l2h[...], w3_ref[g], preferred_element_type=jnp.float32)
        y = jnp.maximum(h + b3, 0.0).astype(_BF16)
        for i in range(_BB):
            base = i * 96 + ga * 10 + gb
            blk = y[base:base + 80].reshape(8, 10, 256)[:, :8, :]
            o_ref[i, :, ga, :, gb * 256:(gb + 1) * 256] = blk


def _conv_call(a0, mask, rhs0, b0, rhs1, b1, w1, b1c, w2, b2c):
    n = a0.shape[0]
    return pl.pallas_call(
        _conv_kernel,
        out_shape=jax.ShapeDtypeStruct((n, 8, 2, 8, 512), _BF16),
        grid=(n // _BB,),
        in_specs=[
            pl.BlockSpec((_BB, 3, 1408), lambda b: (b, 0, 0)),
            pl.BlockSpec((1280, 128), lambda b: (0, 0)),
            pl.BlockSpec((32, 256), lambda b: (0, 0)),
            pl.BlockSpec((1, 128), lambda b: (0, 0)),
            pl.BlockSpec((1152, 256), lambda b: (0, 0)),
            pl.BlockSpec((1, 128), lambda b: (0, 0)),
            pl.BlockSpec((1152, 256), lambda b: (0, 0)),
            pl.BlockSpec((1, 256), lambda b: (0, 0)),
            pl.BlockSpec((4, 1024, 256), lambda b: (0, 0, 0)),
            pl.BlockSpec((1, 256), lambda b: (0, 0)),
        ],
        out_specs=pl.BlockSpec((_BB, 8, 2, 8, 512), lambda b: (b, 0, 0, 0, 0)),
        scratch_shapes=[
            pltpu.VMEM((32, 1280), _BF16),
            pltpu.VMEM((_BB * 1280, 128), jnp.float32),
            pltpu.VMEM((_BB * 4 * _P0R, 128), _BF16),
            pltpu.VMEM((_BB * 272, 1152), _BF16),
            pltpu.VMEM((_BB * 360, 128), jnp.float32),
            pltpu.VMEM((_BB, 4, 96, 128), _BF16),
            pltpu.VMEM((_BB * 72, 1152), _BF16),
            pltpu.VMEM((_BB, 110, 256), _BF16),
            pltpu.VMEM((_BB * 96, 1024), _BF16),
        ],
        compiler_params=_cparams(("parallel",)),
    )(a0, mask, rhs0, b0, rhs1, b1, w1, b1c, w2, b2c)


def _fc_kernel(a_ref, w1_ref, b1_ref, w2_ref, o_ref, acc_ref):
    i = pl.program_id(1)

    @pl.when(i == 0)
    def _():
        acc_ref[...] = jnp.zeros_like(acc_ref)

    upd = jnp.zeros_like(acc_ref)
    for s in range(8):
        upd += jnp.dot(a_ref[:, 0, 0, s, :], w1_ref[s * 512:(s + 1) * 512],
                       preferred_element_type=jnp.float32)
    acc_ref[...] += upd

    @pl.when(i == pl.num_programs(1) - 1)
    def _():
        h = jnp.maximum(acc_ref[...] + b1_ref[...], 0.0).astype(_BF16)
        o_ref[0] = jnp.dot(h, w2_ref[...], preferred_element_type=jnp.float32)


def _fc_call(a, w1, b1, w2):
    mp = a.shape[0]
    tk = 4096
    return pl.pallas_call(
        _fc_kernel,
        out_shape=jax.ShapeDtypeStruct((2, mp, 128), jnp.float32),
        grid=(2, 16),
        in_specs=[
            pl.BlockSpec((mp, 1, 1, 8, 512),
                         lambda j, i: (0, i // 2, i % 2, 0, 0)),
            pl.BlockSpec((tk, 256), lambda j, i: (i, j)),
            pl.BlockSpec((1, 256), lambda j, i: (0, j)),
            pl.BlockSpec((256, 128), lambda j, i: (j, 0)),
        ],
        out_specs=pl.BlockSpec((1, mp, 128), lambda j, i: (j, 0, 0)),
        scratch_shapes=[pltpu.VMEM((mp, 256), jnp.float32)],
        compiler_params=_cparams(("parallel", "arbitrary")),
    )(a, w1, b1, w2)


def kernel(x, init_w, init_b, layer0_w, layer0_b, layer1_w, layer1_b,
           layer2_w, layer2_b, fc1_w, fc1_b, fc2_w, fc2_b):
    n = x.shape[0]

    xb = jnp.pad(x.astype(_BF16), ((0, 0), (0, 0), (2, 2), (2, 2)))
    a0 = jnp.pad(xb.reshape(n, 3, 1296), ((0, 0), (0, 0), (0, 112)))

    r = jnp.arange(1280)
    u, v = r // 36, r % 36
    ok = (u >= 1) & (u <= 32) & (v >= 1) & (v <= 32)
    mask = jnp.broadcast_to(ok[:, None], (1280, 128)).astype(jnp.float32)

    w27 = init_w.reshape(9, 128, 128)[:, :3, :].reshape(27, 128)
    k27 = jnp.arange(27)[:, None]
    rhs0 = jnp.concatenate(
        [jnp.where(k27 < 15, w27, 0), jnp.where(k27 >= 15, w27, 0)], axis=1)
    rhs0 = jnp.pad(rhs0, ((0, 5), (0, 0)))

    w0 = layer0_w[0]
    kk = jnp.arange(1152)[:, None]
    rhs1 = jnp.concatenate(
        [jnp.where(kk < 640, w0, 0), jnp.where(kk >= 640, w0, 0)], axis=1)

    y3 = _conv_call(a0, mask, rhs0, init_b, rhs1, layer0_b,
                    layer1_w[0], layer1_b, layer2_w, layer2_b)

    parts = _fc_call(y3, fc1_w, fc1_b, fc2_w)
    out = parts[0] + parts[1] + fc2_b
    return out[:, :10]
